# baseline (speedup 1.0000x reference)
import functools

import jax
import jax.numpy as jnp
from jax import lax
from jax.experimental import pallas as pl
from jax.experimental.pallas import tpu as pltpu
from jax.experimental.pallas import tpu_sc as plsc

DIM = 8192
CELLS = 576
LEVELS = 256
BINS = 9
PAIRS = CELLS * BINS
CHUNK = 2048
D_STEPS = DIM // CHUNK

NWORK = 32
ROWS_PER_W = LEVELS // NWORK
PAIRS_SC = 5632
P_PER_W = PAIRS_SC // NWORK

_SC_MESH = plsc.VectorSubcoreMesh(core_axis_name="c", subcore_axis_name="s")


def _sc_counts_body(mag_hbm, out_hbm, row_v, cnt_v, tmp_v):
    wid = lax.axis_index("s") * 2 + lax.axis_index("c")
    lane = lax.iota(jnp.int32, 16)
    acc_out = jnp.zeros((16,), jnp.float32)
    for r in range(ROWS_PER_W):
        pltpu.sync_copy(mag_hbm.at[wid * ROWS_PER_W + r], row_v)

        def body(i, acc):
            chunk = row_v[pl.ds(i * 16, 16)]
            return acc + jnp.where(chunk > 0, 1.0, 0.0)

        acc16 = lax.fori_loop(0, DIM // 16, body, jnp.zeros((16,), jnp.float32))
        for sh in (8, 4, 2, 1):
            tmp_v[...] = acc16
            acc16 = acc16 + plsc.load_gather(tmp_v, [lane ^ sh])
        acc_out = jnp.where(lane == r, acc16, acc_out)
    cnt_v[...] = acc_out
    pltpu.sync_copy(cnt_v, out_hbm.at[wid])


def _sc_gather_body(xf_hbm, cnts_hbm, out_hbm, x_v, cnt_v, c_v):
    wid = lax.axis_index("s") * 2 + lax.axis_index("c")
    base = wid * P_PER_W
    pltpu.sync_copy(xf_hbm.at[pl.ds(base, P_PER_W)], x_v)
    pltpu.sync_copy(cnts_hbm, cnt_v)
    for i in range(P_PER_W // 16):
        xi = x_v[pl.ds(i * 16, 16)]
        idxf = jnp.clip(xi * float(LEVELS - 1) + 0.5, 0.0, float(LEVELS - 1))
        idx = idxf.astype(jnp.int32)
        gidx = ((idx >> 3) << 4) + (idx & 7)
        c_v[pl.ds(i * 16, 16)] = plsc.load_gather(cnt_v, [gidx])
    pltpu.sync_copy(c_v, out_hbm.at[pl.ds(base, P_PER_W)])


@functools.partial(
    pl.kernel,
    out_type=jax.ShapeDtypeStruct((NWORK, 16), jnp.float32),
    mesh=_SC_MESH,
    compiler_params=pltpu.CompilerParams(needs_layout_passes=False),
    scratch_types=[
        pltpu.VMEM((DIM,), jnp.float32),
        pltpu.VMEM((16,), jnp.float32),
        pltpu.VMEM((16,), jnp.float32),
    ],
)
def _sc_counts(mag_hbm, out_hbm, row_v, cnt_v, tmp_v):
    _sc_counts_body(mag_hbm, out_hbm, row_v, cnt_v, tmp_v)


@functools.partial(
    pl.kernel,
    out_type=jax.ShapeDtypeStruct((PAIRS_SC,), jnp.float32),
    mesh=_SC_MESH,
    compiler_params=pltpu.CompilerParams(needs_layout_passes=False),
    scratch_types=[
        pltpu.VMEM((P_PER_W,), jnp.float32),
        pltpu.VMEM((NWORK * 16,), jnp.float32),
        pltpu.VMEM((P_PER_W,), jnp.float32),
    ],
)
def _sc_gather(xf_hbm, cnts_hbm, out_hbm, x_v, cnt_v, c_v):
    _sc_gather_body(xf_hbm, cnts_hbm, out_hbm, x_v, cnt_v, c_v)


SUB = 256


def _ones_dot(x):
    ones = jnp.ones((1, CELLS), jnp.bfloat16)
    return lax.dot_general(ones, x, (((1,), (0,)), ((), ())),
                           preferred_element_type=jnp.float32)


def _main_body(cth_ref, ori_ref, cw_ref, am_ref, out_ref):
    j = pl.program_id(0)
    cw = cw_ref[...]
    ori = ori_ref[...]
    cth = cth_ref[...]
    dvec = lax.broadcasted_iota(jnp.int32, (1, SUB), 1).astype(jnp.bfloat16)
    mats = []
    for sub in range(CHUNK // SUB):
        off = j * CHUNK + sub * SUB
        cws = cw[:, sub * SUB:(sub + 1) * SUB]
        oris = ori[:, sub * SUB:(sub + 1) * SUB]
        acc = jnp.zeros((1, SUB), jnp.float32)
        for b in range(BINS):
            crel = jnp.clip(cth[:, b:b + 1] - jnp.float32(off), 0.0, float(SUB))
            mask = dvec < crel.astype(jnp.bfloat16)
            a_b = _ones_dot(jnp.where(mask, cws, jnp.bfloat16(0)))
            acc = acc + oris[b:b + 1, :] * a_b
        s_col = jnp.sum(oris, axis=0, keepdims=True)
        c_col = _ones_dot(cws)
        mats.append(2.0 * acc - s_col * c_col)
    mat = jnp.concatenate(mats, axis=1)
    am = am_ref[...].astype(jnp.float32)
    partial = jnp.sum(am * mat, axis=1, keepdims=True)

    @pl.when(j == 0)
    def _():
        out_ref[...] = partial

    @pl.when(j > 0)
    def _():
        out_ref[...] = out_ref[...] + partial


def kernel(x, mag_table, ori_w, cell_w, am_w):
    num_classes = am_w.shape[0]
    xf = jnp.pad(jnp.reshape(x, (PAIRS,)), (0, PAIRS_SC - PAIRS))
    cnts = _sc_counts(mag_table)
    c = _sc_gather(xf, jnp.reshape(cnts, (NWORK * 16,)))
    cth = jnp.reshape(c[:PAIRS], (CELLS, BINS))
    cw_bf = cell_w.astype(jnp.bfloat16)
    am_bf = am_w.astype(jnp.bfloat16)
    scores = pl.pallas_call(
        _main_body,
        grid=(D_STEPS,),
        in_specs=[
            pl.BlockSpec((CELLS, BINS), lambda j: (0, 0)),
            pl.BlockSpec((BINS, CHUNK), lambda j: (0, j)),
            pl.BlockSpec((CELLS, CHUNK), lambda j: (0, j)),
            pl.BlockSpec((num_classes, CHUNK), lambda j: (0, j)),
        ],
        out_specs=pl.BlockSpec((num_classes, 1), lambda j: (0, 0)),
        out_shape=jax.ShapeDtypeStruct((num_classes, 1), jnp.float32),
    )(cth, ori_w, cw_bf, am_bf)
    return jnp.reshape(scores, (num_classes,))


# speedup vs baseline: 1.4206x; 1.4206x over previous
import functools

import jax
import jax.numpy as jnp
from jax import lax
from jax.experimental import pallas as pl
from jax.experimental.pallas import tpu as pltpu
from jax.experimental.pallas import tpu_sc as plsc

DIM = 8192
CELLS = 576
LEVELS = 256
BINS = 9
PAIRS = CELLS * BINS
CHUNK = 512
D_STEPS = DIM // CHUNK

NWORK = 32
ROWS_PER_W = LEVELS // NWORK
PAIRS_SC = 5632
P_PER_W = PAIRS_SC // NWORK

_SC_MESH = plsc.VectorSubcoreMesh(core_axis_name="c", subcore_axis_name="s")


def _sc_counts_body(mag_hbm, out_hbm, row_v, cnt_v, tmp_v, sem0, sem1):
    wid = lax.axis_index("s") * 2 + lax.axis_index("c")
    lane = lax.iota(jnp.int32, 16)
    acc_out = jnp.zeros((16,), jnp.float32)
    unroll = 8
    sems = (sem0, sem1)
    copies = [None, None]
    copies[0] = pltpu.async_copy(mag_hbm.at[wid * ROWS_PER_W], row_v.at[0], sems[0])
    for r in range(ROWS_PER_W):
        cur = r % 2
        copies[cur].wait()
        if r + 1 < ROWS_PER_W:
            copies[1 - cur] = pltpu.async_copy(
                mag_hbm.at[wid * ROWS_PER_W + r + 1], row_v.at[1 - cur], sems[1 - cur])

        def body(i, accs):
            return tuple(
                accs[u] + row_v[cur, pl.ds((i * unroll + u) * 16, 16)]
                for u in range(unroll)
            )

        accs = lax.fori_loop(0, DIM // 16 // unroll, body,
                             tuple(jnp.zeros((16,), jnp.float32) for _ in range(unroll)))
        acc16 = (((accs[0] + accs[1]) + (accs[2] + accs[3]))
                 + ((accs[4] + accs[5]) + (accs[6] + accs[7])))
        for sh in (8, 4, 2, 1):
            tmp_v[...] = acc16
            acc16 = acc16 + plsc.load_gather(tmp_v, [lane ^ sh])
        cnt = (acc16 + float(DIM)) * 0.5
        acc_out = jnp.where(lane == r, cnt, acc_out)
    cnt_v[...] = acc_out
    pltpu.sync_copy(cnt_v, out_hbm.at[wid])


def _sc_gather_body(xf_hbm, cnts_hbm, out_hbm, x_v, cnt_v, c_v):
    wid = lax.axis_index("s") * 2 + lax.axis_index("c")
    base = wid * P_PER_W
    pltpu.sync_copy(xf_hbm.at[pl.ds(base, P_PER_W)], x_v)
    pltpu.sync_copy(cnts_hbm, cnt_v)
    for i in range(P_PER_W // 16):
        xi = x_v[pl.ds(i * 16, 16)]
        idxf = jnp.clip(xi * float(LEVELS - 1) + 0.5, 0.0, float(LEVELS - 1))
        idx = idxf.astype(jnp.int32)
        gidx = ((idx >> 3) << 4) + (idx & 7)
        c_v[pl.ds(i * 16, 16)] = plsc.load_gather(cnt_v, [gidx])
    pltpu.sync_copy(c_v, out_hbm.at[pl.ds(base, P_PER_W)])


@functools.partial(
    pl.kernel,
    out_type=jax.ShapeDtypeStruct((NWORK, 16), jnp.float32),
    mesh=_SC_MESH,
    compiler_params=pltpu.CompilerParams(needs_layout_passes=False),
    scratch_types=[
        pltpu.VMEM((2, DIM), jnp.float32),
        pltpu.VMEM((16,), jnp.float32),
        pltpu.VMEM((16,), jnp.float32),
        pltpu.SemaphoreType.DMA,
        pltpu.SemaphoreType.DMA,
    ],
)
def _sc_counts(mag_hbm, out_hbm, row_v, cnt_v, tmp_v, sem0, sem1):
    _sc_counts_body(mag_hbm, out_hbm, row_v, cnt_v, tmp_v, sem0, sem1)


@functools.partial(
    pl.kernel,
    out_type=jax.ShapeDtypeStruct((PAIRS_SC,), jnp.float32),
    mesh=_SC_MESH,
    compiler_params=pltpu.CompilerParams(needs_layout_passes=False),
    scratch_types=[
        pltpu.VMEM((P_PER_W,), jnp.float32),
        pltpu.VMEM((NWORK * 16,), jnp.float32),
        pltpu.VMEM((P_PER_W,), jnp.float32),
    ],
)
def _sc_gather(xf_hbm, cnts_hbm, out_hbm, x_v, cnt_v, c_v):
    _sc_gather_body(xf_hbm, cnts_hbm, out_hbm, x_v, cnt_v, c_v)


def _main_body(cth_ref, ori_ref, cw_ref, am_ref, out_ref):
    j = pl.program_id(0)
    dvec = (lax.broadcasted_iota(jnp.int32, (1, CHUNK), 1)
            + j * CHUNK).astype(jnp.float32)
    cw = cw_ref[...]
    ori = ori_ref[...]
    cth = cth_ref[...]
    acc = jnp.zeros((1, CHUNK), jnp.float32)
    for b in range(BINS):
        mask = dvec < cth[:, b:b + 1]
        a_b = jnp.sum(jnp.where(mask, cw, 0.0), axis=0, keepdims=True)
        acc = acc + ori[b:b + 1, :] * a_b
    s_col = jnp.sum(ori, axis=0, keepdims=True)
    c_col = jnp.sum(cw, axis=0, keepdims=True)
    mat = 2.0 * acc - s_col * c_col
    partial = jnp.sum(am_ref[...] * mat, axis=1, keepdims=True)

    @pl.when(j == 0)
    def _():
        out_ref[...] = partial

    @pl.when(j > 0)
    def _():
        out_ref[...] = out_ref[...] + partial


def kernel(x, mag_table, ori_w, cell_w, am_w):
    num_classes = am_w.shape[0]
    xf = jnp.pad(jnp.reshape(x, (PAIRS,)), (0, PAIRS_SC - PAIRS))
    cnts = _sc_counts(mag_table)
    c = _sc_gather(xf, jnp.reshape(cnts, (NWORK * 16,)))
    cth = jnp.reshape(c[:PAIRS], (CELLS, BINS))
    scores = pl.pallas_call(
        _main_body,
        grid=(D_STEPS,),
        in_specs=[
            pl.BlockSpec((CELLS, BINS), lambda j: (0, 0)),
            pl.BlockSpec((BINS, CHUNK), lambda j: (0, j)),
            pl.BlockSpec((CELLS, CHUNK), lambda j: (0, j)),
            pl.BlockSpec((num_classes, CHUNK), lambda j: (0, j)),
        ],
        out_specs=pl.BlockSpec((num_classes, 1), lambda j: (0, 0)),
        out_shape=jax.ShapeDtypeStruct((num_classes, 1), jnp.float32),
    )(cth, ori_w, cell_w, am_w)
    return jnp.reshape(scores, (num_classes,))
